# Initial kernel scaffold; baseline (speedup 1.0000x reference)
import jax
import jax.numpy as jnp
from jax.experimental import pallas as pl


def kernel(x, Wr, Wg, Wu, Wd):
    raise NotImplementedError("write your pallas kernel here")


# speedup vs baseline: 1.1373x; 1.1373x over previous
import functools

import jax
import jax.numpy as jnp
from jax import lax
from jax.experimental import pallas as pl
from jax.experimental.pallas import tpu as pltpu
from jax.experimental.pallas import tpu_sc as plsc

N = 2048
D = 1024
H = 2816
E = 8
K = 2

TM = 256
P = N * K + E * TM
T = P // TM
TH = 1408
HC = H // TH

NC = 2
NS = 16
NW = NC * NS

_SC_MESH = dict(core_axis_name="c", subcore_axis_name="s")


def _router_body(x_ref, wr_ref, posw_ref, gates_ref, eot_ref):
    xf = x_ref[...]
    wr = wr_ref[...]
    lt = lax.dot_general(wr, xf, (((0,), (1,)), ((), ())),
                         preferred_element_type=jnp.float32)
    e_iota = lax.broadcasted_iota(jnp.int32, (E, N), 0)
    m1 = jnp.max(lt, axis=0, keepdims=True)
    i1 = jnp.min(jnp.where(lt == m1, e_iota, E), axis=0, keepdims=True)
    oh1 = e_iota == i1
    lt2 = jnp.where(oh1, -jnp.inf, lt)
    m2 = jnp.max(lt2, axis=0, keepdims=True)
    i2 = jnp.min(jnp.where(lt2 == m2, e_iota, E), axis=0, keepdims=True)
    oh2 = e_iota == i2
    w1 = jax.nn.sigmoid(m1 - m2)
    w2 = jax.nn.sigmoid(m2 - m1)
    maskf = (oh1 | oh2).astype(jnp.float32)
    r_io = lax.broadcasted_iota(jnp.int32, (N, N), 0)
    c_io = lax.broadcasted_iota(jnp.int32, (N, N), 1)
    upper = (r_io < c_io).astype(jnp.float32)
    ranks = lax.dot_general(maskf, upper, (((1,), (0,)), ((), ())),
                            preferred_element_type=jnp.float32)
    counts = jnp.sum(maskf, axis=1, keepdims=True)
    tiles = jnp.floor((counts + (TM - 1)) / TM)
    l8 = (lax.broadcasted_iota(jnp.int32, (E, E), 0)
          > lax.broadcasted_iota(jnp.int32, (E, E), 1)).astype(jnp.float32)
    ct_excl = lax.dot_general(l8, tiles, (((1,), (0,)), ((), ())),
                              preferred_element_type=jnp.float32)
    base = TM * ct_excl
    pos0 = jnp.sum(oh1.astype(jnp.float32) * (ranks + base), axis=0,
                   keepdims=True)
    pos1 = jnp.sum(oh2.astype(jnp.float32) * (ranks + base), axis=0,
                   keepdims=True)
    posw_ref[...] = jnp.concatenate([pos0, pos1], axis=0).astype(jnp.int32)
    gates_ref[...] = jnp.concatenate([w1, w2], axis=0)
    tile_io = lax.broadcasted_iota(jnp.int32, (E, 32), 1).astype(jnp.float32)
    ge = (tile_io >= ct_excl).astype(jnp.float32)
    eot = jnp.clip(jnp.sum(ge, axis=0, keepdims=True) - 1.0, 0, E - 1)
    eot_ref[...] = eot.astype(jnp.int32)


def _router(xf, Wr):
    return pl.pallas_call(
        _router_body,
        out_shape=(
            jax.ShapeDtypeStruct((K, N), jnp.int32),
            jax.ShapeDtypeStruct((K, N), jnp.float32),
            jax.ShapeDtypeStruct((1, 32), jnp.int32),
        ),
    )(xf, Wr)


def _dispatch_body(posw_hbm, gates_hbm, gidx_hbm, rgate_hbm,
                   pos_v, w_v, gid_v, rg_v):
    wid = lax.axis_index("s") * NC + lax.axis_index("c")

    @pl.when(wid == 0)
    def _():
        def zero(k, _):
            gid_v[pl.ds(k * 16, 16)] = jnp.zeros((16,), jnp.int32)
            rg_v[pl.ds(k * 16, 16)] = jnp.zeros((16,), jnp.float32)
            return 0
        lax.fori_loop(0, P // 16, zero, 0)
        for j in range(K):
            pltpu.sync_copy(posw_hbm.at[j], pos_v)
            pltpu.sync_copy(gates_hbm.at[j], w_v)

            def scat(i, _):
                idx = pos_v[pl.ds(i * 16, 16)]
                tok = lax.iota(jnp.int32, 16) + i * 16
                plsc.store_scatter(gid_v, [idx], tok)
                plsc.store_scatter(rg_v, [idx], w_v[pl.ds(i * 16, 16)])
                return 0
            lax.fori_loop(0, N // 16, scat, 0)
        pltpu.sync_copy(gid_v, gidx_hbm)
        pltpu.sync_copy(rg_v, rgate_hbm)


def _dispatch(posw, gates):
    return pl.kernel(
        _dispatch_body,
        out_type=(
            jax.ShapeDtypeStruct((P,), jnp.int32),
            jax.ShapeDtypeStruct((P,), jnp.float32),
        ),
        mesh=plsc.VectorSubcoreMesh(**_SC_MESH),
        compiler_params=pltpu.CompilerParams(needs_layout_passes=False),
        scratch_types=[
            pltpu.VMEM((N,), jnp.int32),
            pltpu.VMEM((N,), jnp.float32),
            pltpu.VMEM((P,), jnp.int32),
            pltpu.VMEM((P,), jnp.float32),
        ],
    )(posw, gates)


_G_CHUNK = 48


def _gather_body(x_hbm, gidx_hbm, xs_hbm, idx_v, rows_v, sem):
    wid = lax.axis_index("s") * NC + lax.axis_index("c")
    for c in range(P // (NW * _G_CHUNK)):
        base = wid * (P // NW) + c * _G_CHUNK
        pltpu.sync_copy(gidx_hbm.at[pl.ds(base, _G_CHUNK)], idx_v)
        pltpu.async_copy(x_hbm.at[idx_v], rows_v, sem).wait()
        pltpu.sync_copy(rows_v, xs_hbm.at[pl.ds(base, _G_CHUNK)])


def _gather(xf, gidx):
    return pl.kernel(
        _gather_body,
        out_type=jax.ShapeDtypeStruct((P, D), jnp.float32),
        mesh=plsc.VectorSubcoreMesh(**_SC_MESH),
        scratch_types=[
            pltpu.VMEM((_G_CHUNK,), jnp.int32),
            pltpu.VMEM((_G_CHUNK, D), jnp.float32),
            pltpu.SemaphoreType.DMA,
        ],
    )(xf, gidx)


def _gemm_body(eot_ref, xs_ref, wg_ref, wu_ref, wd_ref, rg_ref, y_ref,
               acc_ref):
    h = pl.program_id(1)

    @pl.when(h == 0)
    def _():
        acc_ref[...] = jnp.zeros_like(acc_ref)

    xt = xs_ref[...]
    g = jnp.dot(xt, wg_ref[0], preferred_element_type=jnp.float32)
    u = jnp.dot(xt, wu_ref[0], preferred_element_type=jnp.float32)
    gu = g * jax.nn.sigmoid(g) * u
    acc_ref[...] += jnp.dot(gu, wd_ref[0], preferred_element_type=jnp.float32)

    @pl.when(h == HC - 1)
    def _():
        y_ref[...] = acc_ref[...] * rg_ref[...]


def _gemm(eot, xs, Wg, Wu, Wd, rgate_col):
    grid_spec = pltpu.PrefetchScalarGridSpec(
        num_scalar_prefetch=1,
        grid=(T, HC),
        in_specs=[
            pl.BlockSpec((TM, D), lambda t, h, eot: (t, 0)),
            pl.BlockSpec((1, D, TH), lambda t, h, eot: (eot[t], 0, h)),
            pl.BlockSpec((1, D, TH), lambda t, h, eot: (eot[t], 0, h)),
            pl.BlockSpec((1, TH, D), lambda t, h, eot: (eot[t], h, 0)),
            pl.BlockSpec((TM, 1), lambda t, h, eot: (t, 0)),
        ],
        out_specs=pl.BlockSpec((TM, D), lambda t, h, eot: (t, 0)),
        scratch_shapes=[pltpu.VMEM((TM, D), jnp.float32)],
    )
    return pl.pallas_call(
        _gemm_body,
        grid_spec=grid_spec,
        out_shape=jax.ShapeDtypeStruct((P, D), jnp.float32),
    )(eot, xs, Wg, Wu, Wd, rgate_col)


_C_CHUNK = 32


def _combine_body(y_hbm, posw_hbm, out_hbm, p0_v, p1_v, r0_v, r1_v, sem):
    wid = lax.axis_index("s") * NC + lax.axis_index("c")
    for c in range(N // (NW * _C_CHUNK)):
        tbase = wid * (N // NW) + c * _C_CHUNK
        pltpu.sync_copy(posw_hbm.at[0, pl.ds(tbase, _C_CHUNK)], p0_v)
        pltpu.sync_copy(posw_hbm.at[1, pl.ds(tbase, _C_CHUNK)], p1_v)
        cp0 = pltpu.async_copy(y_hbm.at[p0_v], r0_v, sem)
        cp1 = pltpu.async_copy(y_hbm.at[p1_v], r1_v, sem)
        cp0.wait()
        cp1.wait()

        def row(i, _):
            for j in range(0, D, 16):
                r0_v[i, pl.ds(j, 16)] = (r0_v[i, pl.ds(j, 16)]
                                         + r1_v[i, pl.ds(j, 16)])
            return 0
        lax.fori_loop(0, _C_CHUNK, row, 0)
        pltpu.sync_copy(r0_v, out_hbm.at[pl.ds(tbase, _C_CHUNK)])


def _combine(y, posw):
    return pl.kernel(
        _combine_body,
        out_type=jax.ShapeDtypeStruct((N, D), jnp.float32),
        mesh=plsc.VectorSubcoreMesh(**_SC_MESH),
        scratch_types=[
            pltpu.VMEM((_C_CHUNK,), jnp.int32),
            pltpu.VMEM((_C_CHUNK,), jnp.int32),
            pltpu.VMEM((_C_CHUNK, D), jnp.float32),
            pltpu.VMEM((_C_CHUNK, D), jnp.float32),
            pltpu.SemaphoreType.DMA,
        ],
    )(y, posw)


def kernel(x, Wr, Wg, Wu, Wd):
    b, s, d = x.shape
    xf = x.reshape(b * s, d)
    posw, gates, eot32 = _router(xf, Wr)
    gidx, rgate = _dispatch(posw, gates)
    xs = _gather(xf, gidx)
    eot = eot32.reshape(32)[:T]
    y = _gemm(eot, xs, Wg, Wu, Wd, rgate.reshape(P, 1))
    out = _combine(y, posw)
    return out.reshape(b, s, d)


# speedup vs baseline: 1.2837x; 1.1288x over previous
import functools

import jax
import jax.numpy as jnp
from jax import lax
from jax.experimental import pallas as pl
from jax.experimental.pallas import tpu as pltpu
from jax.experimental.pallas import tpu_sc as plsc

N = 2048
D = 1024
H = 2816
E = 8
K = 2

TM = 256
P = N * K + E * TM
T = P // TM
TH = 1408
HC = H // TH

NC = 2
NS = 16
NW = NC * NS

_SC_MESH = dict(core_axis_name="c", subcore_axis_name="s")


def _router_body(x_ref, wr_ref, posw_ref, gates_ref, eot_ref):
    xf = x_ref[...]
    wr = wr_ref[...]
    lt = lax.dot_general(wr, xf, (((0,), (1,)), ((), ())),
                         preferred_element_type=jnp.float32)
    e_iota = lax.broadcasted_iota(jnp.int32, (E, N), 0)
    m1 = jnp.max(lt, axis=0, keepdims=True)
    i1 = jnp.min(jnp.where(lt == m1, e_iota, E), axis=0, keepdims=True)
    oh1 = e_iota == i1
    lt2 = jnp.where(oh1, -jnp.inf, lt)
    m2 = jnp.max(lt2, axis=0, keepdims=True)
    i2 = jnp.min(jnp.where(lt2 == m2, e_iota, E), axis=0, keepdims=True)
    oh2 = e_iota == i2
    w1 = jax.nn.sigmoid(m1 - m2)
    w2 = jax.nn.sigmoid(m2 - m1)
    maskf = (oh1 | oh2).astype(jnp.float32)
    r_io = lax.broadcasted_iota(jnp.int32, (N, N), 0)
    c_io = lax.broadcasted_iota(jnp.int32, (N, N), 1)
    upper = (r_io < c_io).astype(jnp.float32)
    ranks = lax.dot_general(maskf, upper, (((1,), (0,)), ((), ())),
                            preferred_element_type=jnp.float32)
    counts = jnp.sum(maskf, axis=1, keepdims=True)
    tiles = jnp.floor((counts + (TM - 1)) / TM)
    l8 = (lax.broadcasted_iota(jnp.int32, (E, E), 0)
          > lax.broadcasted_iota(jnp.int32, (E, E), 1)).astype(jnp.float32)
    ct_excl = lax.dot_general(l8, tiles, (((1,), (0,)), ((), ())),
                              preferred_element_type=jnp.float32)
    base = TM * ct_excl
    pos0 = jnp.sum(oh1.astype(jnp.float32) * (ranks + base), axis=0,
                   keepdims=True)
    pos1 = jnp.sum(oh2.astype(jnp.float32) * (ranks + base), axis=0,
                   keepdims=True)
    posw_ref[...] = jnp.concatenate([pos0, pos1], axis=0).astype(jnp.int32)
    gates_ref[...] = jnp.concatenate([w1, w2], axis=0)
    tile_io = lax.broadcasted_iota(jnp.int32, (E, 32), 1).astype(jnp.float32)
    ge = (tile_io >= ct_excl).astype(jnp.float32)
    eot = jnp.clip(jnp.sum(ge, axis=0, keepdims=True) - 1.0, 0, E - 1)
    eot_ref[...] = eot.astype(jnp.int32)


def _router(xf, Wr):
    return pl.pallas_call(
        _router_body,
        out_shape=(
            jax.ShapeDtypeStruct((K, N), jnp.int32),
            jax.ShapeDtypeStruct((K, N), jnp.float32),
            jax.ShapeDtypeStruct((1, 32), jnp.int32),
        ),
    )(xf, Wr)


def _dispatch_body(posw_hbm, gates_hbm, gidx_hbm, rgate_hbm,
                   pos_v, w_v, gid_v, rg_v):
    wid = lax.axis_index("s") * NC + lax.axis_index("c")

    @pl.when(wid == 0)
    def _():
        def zero(k, _):
            gid_v[pl.ds(k * 16, 16)] = jnp.zeros((16,), jnp.int32)
            rg_v[pl.ds(k * 16, 16)] = jnp.zeros((16,), jnp.float32)
            return 0
        lax.fori_loop(0, P // 16, zero, 0)
        for j in range(K):
            pltpu.sync_copy(posw_hbm.at[j], pos_v)
            pltpu.sync_copy(gates_hbm.at[j], w_v)

            def scat(i, _):
                idx = pos_v[pl.ds(i * 16, 16)]
                tok = lax.iota(jnp.int32, 16) + i * 16
                plsc.store_scatter(gid_v, [idx], tok)
                plsc.store_scatter(rg_v, [idx], w_v[pl.ds(i * 16, 16)])
                return 0
            lax.fori_loop(0, N // 16, scat, 0)
        pltpu.sync_copy(gid_v, gidx_hbm)
        pltpu.sync_copy(rg_v, rgate_hbm)


def _dispatch(posw, gates):
    return pl.kernel(
        _dispatch_body,
        out_type=(
            jax.ShapeDtypeStruct((P,), jnp.int32),
            jax.ShapeDtypeStruct((P,), jnp.float32),
        ),
        mesh=plsc.VectorSubcoreMesh(**_SC_MESH),
        compiler_params=pltpu.CompilerParams(needs_layout_passes=False),
        scratch_types=[
            pltpu.VMEM((N,), jnp.int32),
            pltpu.VMEM((N,), jnp.float32),
            pltpu.VMEM((P,), jnp.int32),
            pltpu.VMEM((P,), jnp.float32),
        ],
    )(posw, gates)


_G_CHUNK = 48
_G_NCH = P // (NW * _G_CHUNK)


def _gather_body(x_hbm, gidx_hbm, xs_hbm, *args):
    idx_vs = args[:_G_NCH]
    bufs = args[_G_NCH:_G_NCH + 2]
    sem_g, sem_w = args[_G_NCH + 2:]
    wid = lax.axis_index("s") * NC + lax.axis_index("c")
    base = wid * (P // NW)
    for c in range(_G_NCH):
        pltpu.sync_copy(gidx_hbm.at[pl.ds(base + c * _G_CHUNK, _G_CHUNK)],
                        idx_vs[c])
    wb = [None] * _G_NCH
    g_next = pltpu.async_copy(x_hbm.at[idx_vs[0]], bufs[0], sem_g)
    for c in range(_G_NCH):
        g_next.wait()
        if c + 1 < _G_NCH:
            if c >= 1:
                wb[c - 1].wait()
            g_next = pltpu.async_copy(x_hbm.at[idx_vs[c + 1]],
                                      bufs[(c + 1) % 2], sem_g)
        wb[c] = pltpu.async_copy(
            bufs[c % 2], xs_hbm.at[pl.ds(base + c * _G_CHUNK, _G_CHUNK)],
            sem_w)
    wb[_G_NCH - 2].wait()
    wb[_G_NCH - 1].wait()


def _gather(xf, gidx):
    return pl.kernel(
        _gather_body,
        out_type=jax.ShapeDtypeStruct((P, D), jnp.float32),
        mesh=plsc.VectorSubcoreMesh(**_SC_MESH),
        scratch_types=[pltpu.VMEM((_G_CHUNK,), jnp.int32)] * _G_NCH
        + [pltpu.VMEM((_G_CHUNK, D), jnp.float32)] * 2
        + [pltpu.SemaphoreType.DMA, pltpu.SemaphoreType.DMA],
    )(xf, gidx)


GROUP = 8
NG = T // GROUP


def _gemm_body(eot_ref, xs_ref, wg_ref, wu_ref, wd_ref, rg_ref, y_ref,
               acc_ref):
    h = pl.program_id(1)
    ts = pl.program_id(2)

    xt = xs_ref[...]
    g = jnp.dot(xt, wg_ref[0], preferred_element_type=jnp.float32)
    u = jnp.dot(xt, wu_ref[0], preferred_element_type=jnp.float32)
    gu = g * jax.nn.sigmoid(g) * u
    part = jnp.dot(gu, wd_ref[0], preferred_element_type=jnp.float32)

    @pl.when(h == 0)
    def _():
        acc_ref[ts] = part

    @pl.when(h == HC - 1)
    def _():
        y_ref[...] = (acc_ref[ts] + part) * rg_ref[...]


def _gemm(eot, xs, Wg, Wu, Wd, rgate_col):
    grid_spec = pltpu.PrefetchScalarGridSpec(
        num_scalar_prefetch=1,
        grid=(NG, HC, GROUP),
        in_specs=[
            pl.BlockSpec((TM, D), lambda tg, h, ts, eot: (tg * GROUP + ts, 0)),
            pl.BlockSpec((1, D, TH),
                         lambda tg, h, ts, eot: (eot[tg * GROUP + ts], 0, h)),
            pl.BlockSpec((1, D, TH),
                         lambda tg, h, ts, eot: (eot[tg * GROUP + ts], 0, h)),
            pl.BlockSpec((1, TH, D),
                         lambda tg, h, ts, eot: (eot[tg * GROUP + ts], h, 0)),
            pl.BlockSpec((TM, 1), lambda tg, h, ts, eot: (tg * GROUP + ts, 0)),
        ],
        out_specs=pl.BlockSpec(
            (TM, D),
            lambda tg, h, ts, eot: (jnp.where(h == HC - 1,
                                              tg * GROUP + ts, T), 0)),
        scratch_shapes=[pltpu.VMEM((GROUP, TM, D), jnp.float32)],
    )
    return pl.pallas_call(
        _gemm_body,
        grid_spec=grid_spec,
        out_shape=jax.ShapeDtypeStruct((P + TM, D), jnp.float32),
    )(eot, xs, Wg, Wu, Wd, rgate_col)


_C_CHUNK = 32


def _combine_body(y_hbm, posw_hbm, out_hbm, p0_v, p1_v, r0_v, r1_v, sem):
    wid = lax.axis_index("s") * NC + lax.axis_index("c")
    for c in range(N // (NW * _C_CHUNK)):
        tbase = wid * (N // NW) + c * _C_CHUNK
        pltpu.sync_copy(posw_hbm.at[0, pl.ds(tbase, _C_CHUNK)], p0_v)
        pltpu.sync_copy(posw_hbm.at[1, pl.ds(tbase, _C_CHUNK)], p1_v)
        cp0 = pltpu.async_copy(y_hbm.at[p0_v], r0_v, sem)
        cp1 = pltpu.async_copy(y_hbm.at[p1_v], r1_v, sem)
        cp0.wait()
        cp1.wait()

        def row(i, _):
            for j in range(0, D, 16):
                r0_v[i, pl.ds(j, 16)] = (r0_v[i, pl.ds(j, 16)]
                                         + r1_v[i, pl.ds(j, 16)])
            return 0
        lax.fori_loop(0, _C_CHUNK, row, 0)
        pltpu.sync_copy(r0_v, out_hbm.at[pl.ds(tbase, _C_CHUNK)])


def _combine(y, posw):
    return pl.kernel(
        _combine_body,
        out_type=jax.ShapeDtypeStruct((N, D), jnp.float32),
        mesh=plsc.VectorSubcoreMesh(**_SC_MESH),
        scratch_types=[
            pltpu.VMEM((_C_CHUNK,), jnp.int32),
            pltpu.VMEM((_C_CHUNK,), jnp.int32),
            pltpu.VMEM((_C_CHUNK, D), jnp.float32),
            pltpu.VMEM((_C_CHUNK, D), jnp.float32),
            pltpu.SemaphoreType.DMA,
        ],
    )(y, posw)


def kernel(x, Wr, Wg, Wu, Wd):
    b, s, d = x.shape
    xf = x.reshape(b * s, d)
    posw, gates, eot32 = _router(xf, Wr)
    gidx, rgate = _dispatch(posw, gates)
    xs = _gather(xf, gidx)
    eot = eot32.reshape(32)[:T]
    y = _gemm(eot, xs, Wg, Wu, Wd, rgate.reshape(P, 1))
    out = _combine(y, posw)
    return out.reshape(b, s, d)


# speedup vs baseline: 1.3476x; 1.0498x over previous
import functools

import jax
import jax.numpy as jnp
from jax import lax
from jax.experimental import pallas as pl
from jax.experimental.pallas import tpu as pltpu
from jax.experimental.pallas import tpu_sc as plsc

N = 2048
D = 1024
H = 2816
E = 8
K = 2

TM = 256
P = N * K + E * TM
T = P // TM
TH = 1408
HC = H // TH

NC = 2
NS = 16
NW = NC * NS

_SC_MESH = dict(core_axis_name="c", subcore_axis_name="s")


def _router_body(x_ref, wr_ref, posw_ref, gates_ref, eot_ref):
    xf = x_ref[...]
    wr = wr_ref[...]
    lt = lax.dot_general(wr, xf, (((0,), (1,)), ((), ())),
                         preferred_element_type=jnp.float32)
    e_iota = lax.broadcasted_iota(jnp.int32, (E, N), 0)
    m1 = jnp.max(lt, axis=0, keepdims=True)
    i1 = jnp.min(jnp.where(lt == m1, e_iota, E), axis=0, keepdims=True)
    oh1 = e_iota == i1
    lt2 = jnp.where(oh1, -jnp.inf, lt)
    m2 = jnp.max(lt2, axis=0, keepdims=True)
    i2 = jnp.min(jnp.where(lt2 == m2, e_iota, E), axis=0, keepdims=True)
    oh2 = e_iota == i2
    w1 = jax.nn.sigmoid(m1 - m2)
    w2 = jax.nn.sigmoid(m2 - m1)
    maskf = (oh1 | oh2).astype(jnp.float32)
    r_io = lax.broadcasted_iota(jnp.int32, (N, N), 0)
    c_io = lax.broadcasted_iota(jnp.int32, (N, N), 1)
    upper = (r_io < c_io).astype(jnp.float32)
    ranks = lax.dot_general(maskf, upper, (((1,), (0,)), ((), ())),
                            preferred_element_type=jnp.float32)
    counts = jnp.sum(maskf, axis=1, keepdims=True)
    tiles = jnp.floor((counts + (TM - 1)) / TM)
    l8 = (lax.broadcasted_iota(jnp.int32, (E, E), 0)
          > lax.broadcasted_iota(jnp.int32, (E, E), 1)).astype(jnp.float32)
    ct_excl = lax.dot_general(l8, tiles, (((1,), (0,)), ((), ())),
                              preferred_element_type=jnp.float32)
    base = TM * ct_excl
    pos0 = jnp.sum(oh1.astype(jnp.float32) * (ranks + base), axis=0,
                   keepdims=True)
    pos1 = jnp.sum(oh2.astype(jnp.float32) * (ranks + base), axis=0,
                   keepdims=True)
    posw_ref[...] = jnp.concatenate([pos0, pos1], axis=0).astype(jnp.int32)
    gates_ref[...] = jnp.concatenate([w1, w2], axis=0)
    tile_io = lax.broadcasted_iota(jnp.int32, (E, 32), 1).astype(jnp.float32)
    ge = (tile_io >= ct_excl).astype(jnp.float32)
    eot = jnp.clip(jnp.sum(ge, axis=0, keepdims=True) - 1.0, 0, E - 1)
    eot_ref[...] = eot.astype(jnp.int32)


def _router(xf, Wr):
    return pl.pallas_call(
        _router_body,
        out_shape=(
            jax.ShapeDtypeStruct((K, N), jnp.int32),
            jax.ShapeDtypeStruct((K, N), jnp.float32),
            jax.ShapeDtypeStruct((1, 32), jnp.int32),
        ),
    )(xf, Wr)


def _dispatch_body(posw_hbm, gates_hbm, gidx_hbm, rgate_hbm,
                   pos_v, w_v, gid_v, rg_v):
    wid = lax.axis_index("s") * NC + lax.axis_index("c")

    @pl.when(wid == 0)
    def _():
        def zero(k, _):
            gid_v[pl.ds(k * 16, 16)] = jnp.zeros((16,), jnp.int32)
            rg_v[pl.ds(k * 16, 16)] = jnp.zeros((16,), jnp.float32)
            return 0
        lax.fori_loop(0, P // 16, zero, 0)
        for j in range(K):
            pltpu.sync_copy(posw_hbm.at[j], pos_v)
            pltpu.sync_copy(gates_hbm.at[j], w_v)

            def scat(i, _):
                idx = pos_v[pl.ds(i * 16, 16)]
                tok = lax.iota(jnp.int32, 16) + i * 16
                plsc.store_scatter(gid_v, [idx], tok)
                plsc.store_scatter(rg_v, [idx], w_v[pl.ds(i * 16, 16)])
                return 0
            lax.fori_loop(0, N // 16, scat, 0)
        pltpu.sync_copy(gid_v, gidx_hbm)
        pltpu.sync_copy(rg_v, rgate_hbm)


def _dispatch(posw, gates):
    return pl.kernel(
        _dispatch_body,
        out_type=(
            jax.ShapeDtypeStruct((P,), jnp.int32),
            jax.ShapeDtypeStruct((P,), jnp.float32),
        ),
        mesh=plsc.VectorSubcoreMesh(**_SC_MESH),
        compiler_params=pltpu.CompilerParams(needs_layout_passes=False),
        scratch_types=[
            pltpu.VMEM((N,), jnp.int32),
            pltpu.VMEM((N,), jnp.float32),
            pltpu.VMEM((P,), jnp.int32),
            pltpu.VMEM((P,), jnp.float32),
        ],
    )(posw, gates)


GROUP = 8
NG = T // GROUP


def _gemm_body(eot_ref, gidx_ref, x_ref, wg_ref, wu_ref, wd_ref, rg_ref,
               y_ref, acc_ref, xg_ref):
    h = pl.program_id(1)
    ts = pl.program_id(2)

    @pl.when(h == 0)
    def _():
        gid = gidx_ref[...]
        tok = lax.broadcasted_iota(jnp.int32, (TM, N), 1)
        onehot = (tok == gid).astype(jnp.bfloat16)
        xg = jnp.dot(onehot, x_ref[...], preferred_element_type=jnp.float32)
        xg_ref[ts] = xg.astype(jnp.bfloat16)

    xt = xg_ref[ts]
    g = jnp.dot(xt, wg_ref[0], preferred_element_type=jnp.float32)
    u = jnp.dot(xt, wu_ref[0], preferred_element_type=jnp.float32)
    gu = g * jax.nn.sigmoid(g) * u
    part = jnp.dot(gu.astype(jnp.bfloat16), wd_ref[0],
                   preferred_element_type=jnp.float32)

    @pl.when(h == 0)
    def _():
        acc_ref[ts] = part

    @pl.when(h == HC - 1)
    def _():
        y_ref[...] = (acc_ref[ts] + part) * rg_ref[...]


def _gemm(eot, gidx_col, xbf, Wg, Wu, Wd, rgate_col):
    grid_spec = pltpu.PrefetchScalarGridSpec(
        num_scalar_prefetch=1,
        grid=(NG, HC, GROUP),
        in_specs=[
            pl.BlockSpec((TM, 1), lambda tg, h, ts, eot: (tg * GROUP + ts, 0)),
            pl.BlockSpec((N, D), lambda tg, h, ts, eot: (0, 0)),
            pl.BlockSpec((1, D, TH),
                         lambda tg, h, ts, eot: (eot[tg * GROUP + ts], 0, h)),
            pl.BlockSpec((1, D, TH),
                         lambda tg, h, ts, eot: (eot[tg * GROUP + ts], 0, h)),
            pl.BlockSpec((1, TH, D),
                         lambda tg, h, ts, eot: (eot[tg * GROUP + ts], h, 0)),
            pl.BlockSpec((TM, 1), lambda tg, h, ts, eot: (tg * GROUP + ts, 0)),
        ],
        out_specs=pl.BlockSpec(
            (TM, D),
            lambda tg, h, ts, eot: (jnp.where(h == HC - 1,
                                              tg * GROUP + ts, T), 0)),
        scratch_shapes=[pltpu.VMEM((GROUP, TM, D), jnp.float32),
                        pltpu.VMEM((GROUP, TM, D), jnp.bfloat16)],
    )
    return pl.pallas_call(
        _gemm_body,
        grid_spec=grid_spec,
        out_shape=jax.ShapeDtypeStruct((P + TM, D), jnp.float32),
    )(eot, gidx_col, xbf, Wg, Wu, Wd, rgate_col)


_C_CHUNK = 32


def _combine_body(y_hbm, posw_hbm, out_hbm, p0_v, p1_v, r0_v, r1_v, sem):
    wid = lax.axis_index("s") * NC + lax.axis_index("c")
    for c in range(N // (NW * _C_CHUNK)):
        tbase = wid * (N // NW) + c * _C_CHUNK
        pltpu.sync_copy(posw_hbm.at[0, pl.ds(tbase, _C_CHUNK)], p0_v)
        pltpu.sync_copy(posw_hbm.at[1, pl.ds(tbase, _C_CHUNK)], p1_v)
        cp0 = pltpu.async_copy(y_hbm.at[p0_v], r0_v, sem)
        cp1 = pltpu.async_copy(y_hbm.at[p1_v], r1_v, sem)
        cp0.wait()
        cp1.wait()

        def row(i, _):
            for j in range(0, D, 16):
                r0_v[i, pl.ds(j, 16)] = (r0_v[i, pl.ds(j, 16)]
                                         + r1_v[i, pl.ds(j, 16)])
            return 0
        lax.fori_loop(0, _C_CHUNK, row, 0)
        pltpu.sync_copy(r0_v, out_hbm.at[pl.ds(tbase, _C_CHUNK)])


def _combine(y, posw):
    return pl.kernel(
        _combine_body,
        out_type=jax.ShapeDtypeStruct((N, D), jnp.float32),
        mesh=plsc.VectorSubcoreMesh(**_SC_MESH),
        scratch_types=[
            pltpu.VMEM((_C_CHUNK,), jnp.int32),
            pltpu.VMEM((_C_CHUNK,), jnp.int32),
            pltpu.VMEM((_C_CHUNK, D), jnp.float32),
            pltpu.VMEM((_C_CHUNK, D), jnp.float32),
            pltpu.SemaphoreType.DMA,
        ],
    )(y, posw)


def kernel(x, Wr, Wg, Wu, Wd):
    b, s, d = x.shape
    xf = x.reshape(b * s, d)
    posw, gates, eot32 = _router(xf, Wr)
    gidx, rgate = _dispatch(posw, gates)
    eot = eot32.reshape(32)[:T]
    y = _gemm(eot, gidx.reshape(P, 1), xf.astype(jnp.bfloat16),
              Wg.astype(jnp.bfloat16), Wu.astype(jnp.bfloat16),
              Wd.astype(jnp.bfloat16), rgate.reshape(P, 1))
    out = _combine(y, posw)
    return out.reshape(b, s, d)


# speedup vs baseline: 1.8839x; 1.3979x over previous
import functools

import jax
import jax.numpy as jnp
from jax import lax
from jax.experimental import pallas as pl
from jax.experimental.pallas import tpu as pltpu
from jax.experimental.pallas import tpu_sc as plsc

N = 2048
D = 1024
H = 2816
E = 8
K = 2

TM = 256
P = N * K + E * TM
T = P // TM
TH = 1408
HC = H // TH

NC = 2
NS = 16
NW = NC * NS

_SC_MESH = dict(core_axis_name="c", subcore_axis_name="s")


def _router_body(x_ref, wr_ref, posw_ref, gates_ref, eot_ref):
    xf = x_ref[...]
    wr = wr_ref[...]
    lt = lax.dot_general(wr, xf, (((0,), (1,)), ((), ())),
                         preferred_element_type=jnp.float32)
    e_iota = lax.broadcasted_iota(jnp.int32, (E, N), 0)
    m1 = jnp.max(lt, axis=0, keepdims=True)
    i1 = jnp.min(jnp.where(lt == m1, e_iota, E), axis=0, keepdims=True)
    oh1 = e_iota == i1
    lt2 = jnp.where(oh1, -jnp.inf, lt)
    m2 = jnp.max(lt2, axis=0, keepdims=True)
    i2 = jnp.min(jnp.where(lt2 == m2, e_iota, E), axis=0, keepdims=True)
    oh2 = e_iota == i2
    w1 = jax.nn.sigmoid(m1 - m2)
    w2 = jax.nn.sigmoid(m2 - m1)
    maskf = (oh1 | oh2).astype(jnp.float32)
    r_io = lax.broadcasted_iota(jnp.int32, (N, N), 0)
    c_io = lax.broadcasted_iota(jnp.int32, (N, N), 1)
    upper = (r_io < c_io).astype(jnp.float32)
    ranks = lax.dot_general(maskf, upper, (((1,), (0,)), ((), ())),
                            preferred_element_type=jnp.float32)
    counts = jnp.sum(maskf, axis=1, keepdims=True)
    tiles = jnp.floor((counts + (TM - 1)) / TM)
    l8 = (lax.broadcasted_iota(jnp.int32, (E, E), 0)
          > lax.broadcasted_iota(jnp.int32, (E, E), 1)).astype(jnp.float32)
    ct_excl = lax.dot_general(l8, tiles, (((1,), (0,)), ((), ())),
                              preferred_element_type=jnp.float32)
    base = TM * ct_excl
    pos0 = jnp.sum(oh1.astype(jnp.float32) * (ranks + base), axis=0,
                   keepdims=True)
    pos1 = jnp.sum(oh2.astype(jnp.float32) * (ranks + base), axis=0,
                   keepdims=True)
    posw_ref[...] = jnp.concatenate([pos0, pos1], axis=0).astype(jnp.int32)
    gates_ref[...] = jnp.concatenate([w1, w2], axis=0)
    tile_io = lax.broadcasted_iota(jnp.int32, (E, 32), 1).astype(jnp.float32)
    ge = (tile_io >= ct_excl).astype(jnp.float32)
    eot = jnp.clip(jnp.sum(ge, axis=0, keepdims=True) - 1.0, 0, E - 1)
    total_tiles = jnp.sum(tiles)
    act = (lax.broadcasted_iota(jnp.int32, (1, 32), 1).astype(jnp.float32)
           < total_tiles).astype(jnp.float32)
    eot_ref[...] = jnp.concatenate([eot, act], axis=0).astype(jnp.int32)


def _router(xf, Wr):
    return pl.pallas_call(
        _router_body,
        out_shape=(
            jax.ShapeDtypeStruct((K, N), jnp.int32),
            jax.ShapeDtypeStruct((K, N), jnp.float32),
            jax.ShapeDtypeStruct((2, 32), jnp.int32),
        ),
    )(xf, Wr)


def _dispatch_body(posw_hbm, gates_hbm, gidx_hbm, rgate_hbm,
                   pos_v, w_v, gid_v, rg_v):
    wid = lax.axis_index("s") * NC + lax.axis_index("c")

    @pl.when(wid == 0)
    def _():
        def zero(k, _):
            gid_v[pl.ds(k * 16, 16)] = jnp.zeros((16,), jnp.int32)
            rg_v[pl.ds(k * 16, 16)] = jnp.zeros((16,), jnp.float32)
            return 0
        lax.fori_loop(0, P // 16, zero, 0)
        for j in range(K):
            pltpu.sync_copy(posw_hbm.at[j], pos_v)
            pltpu.sync_copy(gates_hbm.at[j], w_v)

            def scat(i, _):
                idx = pos_v[pl.ds(i * 16, 16)]
                tok = lax.iota(jnp.int32, 16) + i * 16
                plsc.store_scatter(gid_v, [idx], tok)
                plsc.store_scatter(rg_v, [idx], w_v[pl.ds(i * 16, 16)])
                return 0
            lax.fori_loop(0, N // 16, scat, 0)
        pltpu.sync_copy(gid_v, gidx_hbm)
        pltpu.sync_copy(rg_v, rgate_hbm)


def _dispatch(posw, gates):
    return pl.kernel(
        _dispatch_body,
        out_type=(
            jax.ShapeDtypeStruct((P,), jnp.int32),
            jax.ShapeDtypeStruct((P,), jnp.float32),
        ),
        mesh=plsc.VectorSubcoreMesh(**_SC_MESH),
        compiler_params=pltpu.CompilerParams(needs_layout_passes=False),
        scratch_types=[
            pltpu.VMEM((N,), jnp.int32),
            pltpu.VMEM((N,), jnp.float32),
            pltpu.VMEM((P,), jnp.int32),
            pltpu.VMEM((P,), jnp.float32),
        ],
    )(posw, gates)


GROUP = 8
NG = T // GROUP


def _gemm_body(meta_ref, gidx_ref, x_ref, wg_ref, wu_ref, wd_ref, rg_ref,
               y_ref, acc_ref, xg_ref):
    h = pl.program_id(1)
    ts = pl.program_id(2)
    tile = pl.program_id(0) * GROUP + ts

    @pl.when(meta_ref[32 + tile] == 1)
    def _():
        @pl.when(h == 0)
        def _():
            gid = gidx_ref[...]
            tok = lax.broadcasted_iota(jnp.int32, (TM, N), 1)
            onehot = (tok == gid).astype(jnp.bfloat16)
            xg = jnp.dot(onehot, x_ref[...],
                         preferred_element_type=jnp.float32)
            xg_ref[ts] = xg.astype(jnp.bfloat16)

        xt = xg_ref[ts]
        wg = wg_ref[0].astype(jnp.bfloat16)
        wu = wu_ref[0].astype(jnp.bfloat16)
        wd = wd_ref[0].astype(jnp.bfloat16)
        g = jnp.dot(xt, wg, preferred_element_type=jnp.float32)
        u = jnp.dot(xt, wu, preferred_element_type=jnp.float32)
        gu = g * jax.nn.sigmoid(g) * u
        part = jnp.dot(gu.astype(jnp.bfloat16), wd,
                       preferred_element_type=jnp.float32)

        @pl.when(h == 0)
        def _():
            acc_ref[ts] = part

        @pl.when(h == HC - 1)
        def _():
            y_ref[...] = (acc_ref[ts] + part) * rg_ref[...]


def _gemm(meta, gidx_col, xbf, Wg, Wu, Wd, rgate_col):
    grid_spec = pltpu.PrefetchScalarGridSpec(
        num_scalar_prefetch=1,
        grid=(NG, HC, GROUP),
        in_specs=[
            pl.BlockSpec((TM, 1), lambda tg, h, ts, m: (tg * GROUP + ts, 0)),
            pl.BlockSpec((N, D), lambda tg, h, ts, m: (0, 0)),
            pl.BlockSpec((1, D, TH),
                         lambda tg, h, ts, m: (m[tg * GROUP + ts], 0, h)),
            pl.BlockSpec((1, D, TH),
                         lambda tg, h, ts, m: (m[tg * GROUP + ts], 0, h)),
            pl.BlockSpec((1, TH, D),
                         lambda tg, h, ts, m: (m[tg * GROUP + ts], h, 0)),
            pl.BlockSpec((TM, 1), lambda tg, h, ts, m: (tg * GROUP + ts, 0)),
        ],
        out_specs=pl.BlockSpec(
            (TM, D),
            lambda tg, h, ts, m: (jnp.where(h == HC - 1,
                                            tg * GROUP + ts, T), 0)),
        scratch_shapes=[pltpu.VMEM((GROUP, TM, D), jnp.float32),
                        pltpu.VMEM((GROUP, TM, D), jnp.bfloat16)],
    )
    return pl.pallas_call(
        _gemm_body,
        grid_spec=grid_spec,
        out_shape=jax.ShapeDtypeStruct((P + TM, D), jnp.float32),
    )(meta, gidx_col, xbf, Wg, Wu, Wd, rgate_col)


_C_CHUNK = 32


def _combine_body(y_hbm, posw_hbm, out_hbm, p0_v, p1_v, r0_v, r1_v, sem):
    wid = lax.axis_index("s") * NC + lax.axis_index("c")
    for c in range(N // (NW * _C_CHUNK)):
        tbase = wid * (N // NW) + c * _C_CHUNK
        pltpu.sync_copy(posw_hbm.at[0, pl.ds(tbase, _C_CHUNK)], p0_v)
        pltpu.sync_copy(posw_hbm.at[1, pl.ds(tbase, _C_CHUNK)], p1_v)
        cp0 = pltpu.async_copy(y_hbm.at[p0_v], r0_v, sem)
        cp1 = pltpu.async_copy(y_hbm.at[p1_v], r1_v, sem)
        cp0.wait()
        cp1.wait()

        def row(i, _):
            for j in range(0, D, 16):
                r0_v[i, pl.ds(j, 16)] = (r0_v[i, pl.ds(j, 16)]
                                         + r1_v[i, pl.ds(j, 16)])
            return 0
        lax.fori_loop(0, _C_CHUNK, row, 0)
        pltpu.sync_copy(r0_v, out_hbm.at[pl.ds(tbase, _C_CHUNK)])


def _combine(y, posw):
    return pl.kernel(
        _combine_body,
        out_type=jax.ShapeDtypeStruct((N, D), jnp.float32),
        mesh=plsc.VectorSubcoreMesh(**_SC_MESH),
        scratch_types=[
            pltpu.VMEM((_C_CHUNK,), jnp.int32),
            pltpu.VMEM((_C_CHUNK,), jnp.int32),
            pltpu.VMEM((_C_CHUNK, D), jnp.float32),
            pltpu.VMEM((_C_CHUNK, D), jnp.float32),
            pltpu.SemaphoreType.DMA,
        ],
    )(y, posw)


def kernel(x, Wr, Wg, Wu, Wd):
    b, s, d = x.shape
    xf = x.reshape(b * s, d)
    posw, gates, meta2 = _router(xf, Wr)
    gidx, rgate = _dispatch(posw, gates)
    meta = meta2.reshape(64)
    y = _gemm(meta, gidx.reshape(P, 1), xf.astype(jnp.bfloat16),
              Wg, Wu, Wd, rgate.reshape(P, 1))
    out = _combine(y, posw)
    return out.reshape(b, s, d)


# speedup vs baseline: 1.9326x; 1.0259x over previous
import functools

import jax
import jax.numpy as jnp
from jax import lax
from jax.experimental import pallas as pl
from jax.experimental.pallas import tpu as pltpu
from jax.experimental.pallas import tpu_sc as plsc

N = 2048
D = 1024
H = 2816
E = 8
K = 2

TM = 256
P = N * K + E * TM
T = P // TM
TH = 1408
HC = H // TH

NC = 2
NS = 16
NW = NC * NS

_SC_MESH = dict(core_axis_name="c", subcore_axis_name="s")


def _router_body(x_ref, wr_ref, posw_ref, gates_ref, eot_ref):
    xf = x_ref[...]
    wr = wr_ref[...]
    lt = lax.dot_general(wr, xf, (((0,), (1,)), ((), ())),
                         preferred_element_type=jnp.float32)
    e_iota = lax.broadcasted_iota(jnp.int32, (E, N), 0)
    m1 = jnp.max(lt, axis=0, keepdims=True)
    i1 = jnp.min(jnp.where(lt == m1, e_iota, E), axis=0, keepdims=True)
    oh1 = e_iota == i1
    lt2 = jnp.where(oh1, -jnp.inf, lt)
    m2 = jnp.max(lt2, axis=0, keepdims=True)
    i2 = jnp.min(jnp.where(lt2 == m2, e_iota, E), axis=0, keepdims=True)
    oh2 = e_iota == i2
    w1 = jax.nn.sigmoid(m1 - m2)
    w2 = jax.nn.sigmoid(m2 - m1)
    maskf = (oh1 | oh2).astype(jnp.float32)
    r_io = lax.broadcasted_iota(jnp.int32, (N, N), 0)
    c_io = lax.broadcasted_iota(jnp.int32, (N, N), 1)
    upper = (r_io < c_io).astype(jnp.float32)
    ranks = lax.dot_general(maskf, upper, (((1,), (0,)), ((), ())),
                            preferred_element_type=jnp.float32)
    counts = jnp.sum(maskf, axis=1, keepdims=True)
    tiles = jnp.floor((counts + (TM - 1)) / TM)
    l8 = (lax.broadcasted_iota(jnp.int32, (E, E), 0)
          > lax.broadcasted_iota(jnp.int32, (E, E), 1)).astype(jnp.float32)
    ct_excl = lax.dot_general(l8, tiles, (((1,), (0,)), ((), ())),
                              preferred_element_type=jnp.float32)
    base = TM * ct_excl
    pos0 = jnp.sum(oh1.astype(jnp.float32) * (ranks + base), axis=0,
                   keepdims=True)
    pos1 = jnp.sum(oh2.astype(jnp.float32) * (ranks + base), axis=0,
                   keepdims=True)
    posw_ref[...] = jnp.concatenate([pos0, pos1], axis=0).astype(jnp.int32)
    gates_ref[...] = jnp.concatenate([w1, w2], axis=0)
    tile_io = lax.broadcasted_iota(jnp.int32, (E, 32), 1).astype(jnp.float32)
    ge = (tile_io >= ct_excl).astype(jnp.float32)
    eot = jnp.clip(jnp.sum(ge, axis=0, keepdims=True) - 1.0, 0, E - 1)
    total_tiles = jnp.sum(tiles)
    act = (lax.broadcasted_iota(jnp.int32, (1, 32), 1).astype(jnp.float32)
           < total_tiles).astype(jnp.float32)
    eot_ref[...] = jnp.concatenate([eot, act], axis=0).astype(jnp.int32)


def _router(xf, Wr):
    return pl.pallas_call(
        _router_body,
        out_shape=(
            jax.ShapeDtypeStruct((K, N), jnp.int32),
            jax.ShapeDtypeStruct((K, N), jnp.float32),
            jax.ShapeDtypeStruct((2, 32), jnp.int32),
        ),
    )(xf, Wr)


GROUP = 8
NG = T // GROUP


def _gemm_body(meta_ref, posw_ref, gates_ref, x_ref, wg_ref, wu_ref, wd_ref,
               y_ref, acc_ref, xg_ref):
    h = pl.program_id(1)
    ts = pl.program_id(2)
    tile = pl.program_id(0) * GROUP + ts

    @pl.when(meta_ref[32 + tile] == 1)
    def _():
        r_glob = (tile * TM
                  + lax.broadcasted_iota(jnp.int32, (TM, N), 0))
        oh0 = r_glob == posw_ref[0:1, :]
        oh1 = r_glob == posw_ref[1:2, :]

        @pl.when(h == 0)
        def _():
            onehot = (oh0 | oh1).astype(jnp.bfloat16)
            xg = jnp.dot(onehot, x_ref[...],
                         preferred_element_type=jnp.float32)
            xg_ref[ts] = xg.astype(jnp.bfloat16)

        xt = xg_ref[ts]
        wg = wg_ref[0].astype(jnp.bfloat16)
        wu = wu_ref[0].astype(jnp.bfloat16)
        wd = wd_ref[0].astype(jnp.bfloat16)
        g = jnp.dot(xt, wg, preferred_element_type=jnp.float32)
        u = jnp.dot(xt, wu, preferred_element_type=jnp.float32)
        gu = g * jax.nn.sigmoid(g) * u
        part = jnp.dot(gu.astype(jnp.bfloat16), wd,
                       preferred_element_type=jnp.float32)

        @pl.when(h == 0)
        def _():
            acc_ref[ts] = part

        @pl.when(h == HC - 1)
        def _():
            zero = jnp.zeros((), jnp.float32)
            ohw = (jnp.where(oh0, gates_ref[0:1, :], zero)
                   + jnp.where(oh1, gates_ref[1:2, :], zero))
            rg = jnp.sum(ohw, axis=1, keepdims=True)
            y_ref[...] = (acc_ref[ts] + part) * rg


def _gemm(meta, posw, gates, xbf, Wg, Wu, Wd):
    grid_spec = pltpu.PrefetchScalarGridSpec(
        num_scalar_prefetch=1,
        grid=(NG, HC, GROUP),
        in_specs=[
            pl.BlockSpec((K, N), lambda tg, h, ts, m: (0, 0)),
            pl.BlockSpec((K, N), lambda tg, h, ts, m: (0, 0)),
            pl.BlockSpec((N, D), lambda tg, h, ts, m: (0, 0)),
            pl.BlockSpec((1, D, TH),
                         lambda tg, h, ts, m: (m[tg * GROUP + ts], 0, h)),
            pl.BlockSpec((1, D, TH),
                         lambda tg, h, ts, m: (m[tg * GROUP + ts], 0, h)),
            pl.BlockSpec((1, TH, D),
                         lambda tg, h, ts, m: (m[tg * GROUP + ts], h, 0)),
        ],
        out_specs=pl.BlockSpec(
            (TM, D),
            lambda tg, h, ts, m: (jnp.where(h == HC - 1,
                                            tg * GROUP + ts, T), 0)),
        scratch_shapes=[pltpu.VMEM((GROUP, TM, D), jnp.float32),
                        pltpu.VMEM((GROUP, TM, D), jnp.bfloat16)],
    )
    return pl.pallas_call(
        _gemm_body,
        grid_spec=grid_spec,
        out_shape=jax.ShapeDtypeStruct((P + TM, D), jnp.float32),
    )(meta, posw, gates, xbf, Wg, Wu, Wd)


_C_CHUNK = 32


def _combine_body(y_hbm, posw_hbm, out_hbm, p0_v, p1_v, r0_v, r1_v, sem):
    wid = lax.axis_index("s") * NC + lax.axis_index("c")
    for c in range(N // (NW * _C_CHUNK)):
        tbase = wid * (N // NW) + c * _C_CHUNK
        pltpu.sync_copy(posw_hbm.at[0, pl.ds(tbase, _C_CHUNK)], p0_v)
        pltpu.sync_copy(posw_hbm.at[1, pl.ds(tbase, _C_CHUNK)], p1_v)
        cp0 = pltpu.async_copy(y_hbm.at[p0_v], r0_v, sem)
        cp1 = pltpu.async_copy(y_hbm.at[p1_v], r1_v, sem)
        cp0.wait()
        cp1.wait()

        def row(i, _):
            for j in range(0, D, 16):
                r0_v[i, pl.ds(j, 16)] = (r0_v[i, pl.ds(j, 16)]
                                         + r1_v[i, pl.ds(j, 16)])
            return 0
        lax.fori_loop(0, _C_CHUNK, row, 0)
        pltpu.sync_copy(r0_v, out_hbm.at[pl.ds(tbase, _C_CHUNK)])


def _combine(y, posw):
    return pl.kernel(
        _combine_body,
        out_type=jax.ShapeDtypeStruct((N, D), jnp.float32),
        mesh=plsc.VectorSubcoreMesh(**_SC_MESH),
        scratch_types=[
            pltpu.VMEM((_C_CHUNK,), jnp.int32),
            pltpu.VMEM((_C_CHUNK,), jnp.int32),
            pltpu.VMEM((_C_CHUNK, D), jnp.float32),
            pltpu.VMEM((_C_CHUNK, D), jnp.float32),
            pltpu.SemaphoreType.DMA,
        ],
    )(y, posw)


def kernel(x, Wr, Wg, Wu, Wd):
    b, s, d = x.shape
    xf = x.reshape(b * s, d)
    posw, gates, meta2 = _router(xf, Wr)
    meta = meta2.reshape(64)
    y = _gemm(meta, posw, gates, xf.astype(jnp.bfloat16), Wg, Wu, Wd)
    out = _combine(y, posw)
    return out.reshape(b, s, d)
